# baseline (device time: 108091 ns/iter reference)
import jax
import jax.numpy as jnp
from jax import lax
from jax.experimental import pallas as pl
from jax.experimental.pallas import tpu as pltpu

N_DEV = 8
B, Sq, Skv_shard, Hq, Dh = 2, 256, 256, 4, 64
G = B * Hq
D_MODEL = 512


def kernel(x, Wq, K_ext, V_ext, Wo):
    def body(x_ref, wq_ref, k_ref, v_ref, wo_ref, out_ref,
             ctx_ref, stat_ref, csend_sems, crecv_sems, ssend_sems,
             srecv_sems):
        my = lax.axis_index("i")
        left = lax.rem(my - 1 + N_DEV, N_DEV)
        right = lax.rem(my + 1, N_DEV)

        barrier_sem = pltpu.get_barrier_semaphore()
        for nbr in (left, right):
            pl.semaphore_signal(
                barrier_sem, inc=1,
                device_id=(nbr,), device_id_type=pl.DeviceIdType.MESH,
            )
        pl.semaphore_wait(barrier_sem, 2)

        r = lax.broadcasted_iota(jnp.int32, (Sq, Skv_shard), 0) // 64
        c = lax.broadcasted_iota(jnp.int32, (Sq, Skv_shard), 1) // 64
        mask = r == c

        for b in range(B):
            q_all = jnp.dot(x_ref[b], wq_ref[...],
                            preferred_element_type=jnp.float32)
            for h in range(Hq):
                g = b * Hq + h
                q = q_all[:, h * Dh:(h + 1) * Dh]
                k = k_ref[b, :, h, :]
                s = lax.dot_general(
                    q, k, (((1,), (1,)), ((), ())),
                    preferred_element_type=jnp.float32,
                ) * 0.125
                s = jnp.where(mask, s, -1e9)
                m = jnp.max(s, axis=1)
                p = jnp.exp(s - m[:, None])
                l = jnp.sum(p, axis=1)
                ctx = jnp.dot(p, v_ref[b, :, h, :],
                              preferred_element_type=jnp.float32)
                ctx_ref[0, g] = ctx
                stat_ref[0, 0, g] = m
                stat_ref[0, 1, g] = l

        for t in range(N_DEV - 1):
            rdma_c = pltpu.make_async_remote_copy(
                src_ref=ctx_ref.at[t], dst_ref=ctx_ref.at[t + 1],
                send_sem=csend_sems.at[t], recv_sem=crecv_sems.at[t],
                device_id=(right,), device_id_type=pl.DeviceIdType.MESH,
            )
            rdma_s = pltpu.make_async_remote_copy(
                src_ref=stat_ref.at[t], dst_ref=stat_ref.at[t + 1],
                send_sem=ssend_sems.at[t], recv_sem=srecv_sems.at[t],
                device_id=(right,), device_id_type=pl.DeviceIdType.MESH,
            )
            rdma_c.start()
            rdma_s.start()
            rdma_c.wait()
            rdma_s.wait()

        m_all = stat_ref[:, 0]
        l_all = stat_ref[:, 1]
        m_g = jnp.max(m_all, axis=0)
        w = jnp.exp(m_all - m_g[None])
        l_g = jnp.sum(w * l_all, axis=0)
        ctx_c = jnp.sum(ctx_ref[...] * w[..., None], axis=0)
        ctx_n = ctx_c / l_g[..., None]

        for b in range(B):
            acc = jnp.zeros((Sq, D_MODEL), jnp.float32)
            for h in range(Hq):
                g = b * Hq + h
                acc = acc + jnp.dot(ctx_n[g], wo_ref[h * Dh:(h + 1) * Dh, :],
                                    preferred_element_type=jnp.float32)
            out_ref[b] = acc

    return pl.pallas_call(
        body,
        out_shape=jax.ShapeDtypeStruct((B, Sq, D_MODEL), jnp.float32),
        in_specs=[pl.BlockSpec(memory_space=pltpu.VMEM)] * 5,
        out_specs=pl.BlockSpec(memory_space=pltpu.VMEM),
        scratch_shapes=[
            pltpu.VMEM((N_DEV, G, Sq, Dh), jnp.float32),
            pltpu.VMEM((N_DEV, 2, G, Sq), jnp.float32),
            pltpu.SemaphoreType.DMA((N_DEV - 1,)),
            pltpu.SemaphoreType.DMA((N_DEV - 1,)),
            pltpu.SemaphoreType.DMA((N_DEV - 1,)),
            pltpu.SemaphoreType.DMA((N_DEV - 1,)),
        ],
        compiler_params=pltpu.CompilerParams(collective_id=0),
    )(x, Wq, K_ext, V_ext, Wo)


# device time: 45295 ns/iter; 2.3864x vs baseline; 2.3864x over previous
import jax
import jax.numpy as jnp
from jax import lax
from jax.experimental import pallas as pl
from jax.experimental.pallas import tpu as pltpu

N_DEV = 8
B, Sq, Skv_shard, Hq, Dh = 2, 256, 256, 4, 64
G = B * Hq
D_MODEL = 512
R_HOPS = 4
L_HOPS = 3


def kernel(x, Wq, K_ext, V_ext, Wo):
    def body(x_ref, wq_ref, k_ref, v_ref, wo_ref, out_ref,
             rc_ref, lc_ref, rl_ref, ll_ref, ctx_acc, l_acc,
             rcs, rcr, rls, rlr, lcs, lcr, lls, llr):
        my = lax.axis_index("i")
        left = lax.rem(my - 1 + N_DEV, N_DEV)
        right = lax.rem(my + 1, N_DEV)

        barrier_sem = pltpu.get_barrier_semaphore()
        for nbr in (left, right):
            pl.semaphore_signal(
                barrier_sem, inc=1,
                device_id=(nbr,), device_id_type=pl.DeviceIdType.MESH,
            )
        pl.semaphore_wait(barrier_sem, 2)

        r = lax.broadcasted_iota(jnp.int32, (Sq, Skv_shard), 0) // 64
        c = lax.broadcasted_iota(jnp.int32, (Sq, Skv_shard), 1) // 64
        mask = r == c

        for b in range(B):
            q_all = jnp.dot(x_ref[b], wq_ref[...],
                            preferred_element_type=jnp.float32)
            for h in range(Hq):
                g = b * Hq + h
                q = q_all[:, h * Dh:(h + 1) * Dh]
                k = k_ref[b, :, h, :]
                s = lax.dot_general(
                    q, k, (((1,), (1,)), ((), ())),
                    preferred_element_type=jnp.float32,
                ) * 0.125
                p = jnp.where(mask, jnp.exp(s), 0.0)
                l = jnp.sum(p, axis=1)
                ctx = jnp.dot(p, v_ref[b, :, h, :],
                              preferred_element_type=jnp.float32)
                ctx_acc[g] = ctx
                l_acc[g] = l
                cbf = ctx.astype(jnp.bfloat16)
                rc_ref[0, g] = cbf
                lc_ref[0, g] = cbf
                rl_ref[0, g] = l
                ll_ref[0, g] = l

        def mk(buf, t, sems_s, sems_r, dst):
            return pltpu.make_async_remote_copy(
                src_ref=buf.at[t], dst_ref=buf.at[t + 1],
                send_sem=sems_s.at[t], recv_sem=sems_r.at[t],
                device_id=(dst,), device_id_type=pl.DeviceIdType.MESH,
            )

        descs = []

        def issue(t):
            if t < R_HOPS:
                for d in (mk(rc_ref, t, rcs, rcr, right),
                          mk(rl_ref, t, rls, rlr, right)):
                    d.start()
                    descs.append(d)
            if t < L_HOPS:
                for d in (mk(lc_ref, t, lcs, lcr, left),
                          mk(ll_ref, t, lls, llr, left)):
                    d.start()
                    descs.append(d)

        issue(0)
        for t in range(R_HOPS):
            mk(rc_ref, t, rcs, rcr, right).wait_recv()
            mk(rl_ref, t, rls, rlr, right).wait_recv()
            if t < L_HOPS:
                mk(lc_ref, t, lcs, lcr, left).wait_recv()
                mk(ll_ref, t, lls, llr, left).wait_recv()
            issue(t + 1)
            ctx_acc[...] = ctx_acc[...] + rc_ref[t + 1].astype(jnp.float32)
            l_acc[...] = l_acc[...] + rl_ref[t + 1]
            if t < L_HOPS:
                ctx_acc[...] = ctx_acc[...] + lc_ref[t + 1].astype(jnp.float32)
                l_acc[...] = l_acc[...] + ll_ref[t + 1]

        for d in descs:
            d.wait_send()

        ctx_n = ctx_acc[...] / l_acc[...][..., None]
        for b in range(B):
            acc = jnp.zeros((Sq, D_MODEL), jnp.float32)
            for h in range(Hq):
                g = b * Hq + h
                acc = acc + jnp.dot(ctx_n[g], wo_ref[h * Dh:(h + 1) * Dh, :],
                                    preferred_element_type=jnp.float32)
            out_ref[b] = acc

    return pl.pallas_call(
        body,
        out_shape=jax.ShapeDtypeStruct((B, Sq, D_MODEL), jnp.float32),
        in_specs=[pl.BlockSpec(memory_space=pltpu.VMEM)] * 5,
        out_specs=pl.BlockSpec(memory_space=pltpu.VMEM),
        scratch_shapes=[
            pltpu.VMEM((R_HOPS + 1, G, Sq, Dh), jnp.bfloat16),
            pltpu.VMEM((L_HOPS + 1, G, Sq, Dh), jnp.bfloat16),
            pltpu.VMEM((R_HOPS + 1, G, Sq), jnp.float32),
            pltpu.VMEM((L_HOPS + 1, G, Sq), jnp.float32),
            pltpu.VMEM((G, Sq, Dh), jnp.float32),
            pltpu.VMEM((G, Sq), jnp.float32),
            pltpu.SemaphoreType.DMA((R_HOPS,)),
            pltpu.SemaphoreType.DMA((R_HOPS,)),
            pltpu.SemaphoreType.DMA((R_HOPS,)),
            pltpu.SemaphoreType.DMA((R_HOPS,)),
            pltpu.SemaphoreType.DMA((L_HOPS,)),
            pltpu.SemaphoreType.DMA((L_HOPS,)),
            pltpu.SemaphoreType.DMA((L_HOPS,)),
            pltpu.SemaphoreType.DMA((L_HOPS,)),
        ],
        compiler_params=pltpu.CompilerParams(collective_id=0),
    )(x, Wq, K_ext, V_ext, Wo)


# device time: 22521 ns/iter; 4.7996x vs baseline; 2.0112x over previous
import jax
import jax.numpy as jnp
from jax import lax
from jax.experimental import pallas as pl
from jax.experimental.pallas import tpu as pltpu

N_DEV = 8
B, Sq, Skv_shard, Hq, Dh = 2, 256, 256, 4, 64
G = B * Hq
D_MODEL = 512
STEPS = 3
CTX_ROWS = 512
L_ROWS = 64
HALF_ROWS = CTX_ROWS + L_ROWS


def kernel(x, Wq, K_ext, V_ext, Wo):
    def body(x_ref, wq_ref, k_ref, v_ref, wo_ref, out_ref,
             acc_a, acc_b, rc_a, rc_b, sa_s, sa_r, sb_s, sb_r):
        my = lax.axis_index("i")
        partners = [my ^ 1, my ^ 2, my ^ 4]

        barrier_sem = pltpu.get_barrier_semaphore()
        for p in partners:
            pl.semaphore_signal(
                barrier_sem, inc=1,
                device_id=(p,), device_id_type=pl.DeviceIdType.MESH,
            )
        pl.semaphore_wait(barrier_sem, STEPS)

        r = lax.broadcasted_iota(jnp.int32, (Sq, Skv_shard), 0) // 64
        c = lax.broadcasted_iota(jnp.int32, (Sq, Skv_shard), 1) // 64
        mask = r == c
        wq_s = wq_ref[...] * 0.125

        def compute_half(b, acc):
            q_all = jnp.dot(x_ref[b], wq_s,
                            preferred_element_type=jnp.float32)
            for h in range(Hq):
                jj, side = divmod(h, 2)
                q = q_all[:, h * Dh:(h + 1) * Dh]
                k = k_ref[b, :, h, :]
                s = lax.dot_general(
                    q, k, (((1,), (1,)), ((), ())),
                    preferred_element_type=jnp.float32,
                )
                p = jnp.where(mask, jnp.exp(s), 0.0)
                ctx = jnp.dot(p, v_ref[b, :, h, :],
                              preferred_element_type=jnp.float32)
                acc[jj * Sq:(jj + 1) * Sq,
                    side * Dh:(side + 1) * Dh] = ctx.astype(jnp.bfloat16)
                l = jnp.sum(p, axis=1).astype(jnp.bfloat16)
                for qq in range(4):
                    acc[CTX_ROWS:CTX_ROWS + L_ROWS, 4 * h + qq] = (
                        l[qq * 64:(qq + 1) * 64])

        def start_step(step, acc, rc, s_sems, r_sems):
            rdma = pltpu.make_async_remote_copy(
                src_ref=acc, dst_ref=rc.at[step],
                send_sem=s_sems.at[step], recv_sem=r_sems.at[step],
                device_id=(partners[step],),
                device_id_type=pl.DeviceIdType.MESH,
            )
            rdma.start()
            return rdma

        compute_half(0, acc_a)
        da = start_step(0, acc_a, rc_a, sa_s, sa_r)
        compute_half(1, acc_b)
        db = start_step(0, acc_b, rc_b, sb_s, sb_r)
        for step in range(STEPS):
            da.wait()
            acc_a[...] = acc_a[...] + rc_a[step]
            if step + 1 < STEPS:
                da = start_step(step + 1, acc_a, rc_a, sa_s, sa_r)
            db.wait()
            acc_b[...] = acc_b[...] + rc_b[step]
            if step + 1 < STEPS:
                db = start_step(step + 1, acc_b, rc_b, sb_s, sb_r)

        for b, acc in ((0, acc_a), (1, acc_b)):
            o = jnp.zeros((Sq, D_MODEL), jnp.float32)
            for h in range(Hq):
                jj, side = divmod(h, 2)
                ctx_g = acc[jj * Sq:(jj + 1) * Sq,
                            side * Dh:(side + 1) * Dh].astype(jnp.float32)
                l_g = jnp.concatenate(
                    [acc[CTX_ROWS:CTX_ROWS + L_ROWS, 4 * h + qq][:, None]
                     for qq in range(4)], axis=0).astype(jnp.float32)
                o = o + jnp.dot(ctx_g / l_g,
                                wo_ref[h * Dh:(h + 1) * Dh, :],
                                preferred_element_type=jnp.float32)
            out_ref[b] = o

    return pl.pallas_call(
        body,
        out_shape=jax.ShapeDtypeStruct((B, Sq, D_MODEL), jnp.float32),
        in_specs=[pl.BlockSpec(memory_space=pltpu.VMEM)] * 5,
        out_specs=pl.BlockSpec(memory_space=pltpu.VMEM),
        scratch_shapes=[
            pltpu.VMEM((HALF_ROWS, 128), jnp.bfloat16),
            pltpu.VMEM((HALF_ROWS, 128), jnp.bfloat16),
            pltpu.VMEM((STEPS, HALF_ROWS, 128), jnp.bfloat16),
            pltpu.VMEM((STEPS, HALF_ROWS, 128), jnp.bfloat16),
            pltpu.SemaphoreType.DMA((STEPS,)),
            pltpu.SemaphoreType.DMA((STEPS,)),
            pltpu.SemaphoreType.DMA((STEPS,)),
            pltpu.SemaphoreType.DMA((STEPS,)),
        ],
        compiler_params=pltpu.CompilerParams(collective_id=0),
    )(x, Wq, K_ext, V_ext, Wo)
